# baseline (speedup 1.0000x reference)
import functools

import jax
import jax.numpy as jnp
from jax import lax
from jax.experimental import pallas as pl
from jax.experimental.pallas import tpu as pltpu
from jax.experimental.pallas import tpu_sc as plsc


def _pass_nn(xyz1_ref, xyz2_ref, idx_ref, w_ref, *, m, tn):
    x = xyz1_ref[0]
    y = xyz2_ref[0]
    xb = x[0:3, :].astype(jnp.bfloat16)
    yb = y.astype(jnp.bfloat16)
    cross = jnp.dot(yb, xb, preferred_element_type=jnp.float32)
    sq1 = ((x[0:1, :] * x[0:1, :] + x[1:2, :] * x[1:2, :])
           + x[2:3, :] * x[2:3, :])
    sq2 = ((y[:, 0:1] * y[:, 0:1] + y[:, 1:2] * y[:, 1:2])
           + y[:, 2:3] * y[:, 2:3])
    d2 = (sq1 + sq2) - 2.0 * cross

    iota = jax.lax.broadcasted_iota(jnp.int32, (m, tn), 0)
    dst = []
    idx = []
    for k in range(3):
        mn = jnp.min(d2, axis=0, keepdims=True)
        im = jnp.min(jnp.where(d2 == mn, iota, m), axis=0, keepdims=True)
        dst.append(mn)
        idx.append(im)
        if k < 2:
            d2 = jnp.where(iota == im, jnp.float32(3.4e38), d2)

    rec = [1.0 / jnp.maximum(d, 1e-10) for d in dst]
    norm = rec[0] + rec[1] + rec[2]
    base = pl.program_id(0) * m
    for k in range(3):
        idx_ref[0, k:k + 1, :] = idx[k] + base
        w_ref[0, k:k + 1, :] = rec[k] / norm


def _sc_interp(table, idxg, wexp, bn, n, c2):
    info = plsc.get_sparse_core_info()
    nc = info.num_cores
    nw = nc * info.num_subcores
    ppw = bn // nw
    p = min(128, ppw)
    nchunks = ppw // p
    nch = c2 // 16
    mesh = plsc.VectorSubcoreMesh(core_axis_name="c", subcore_axis_name="s")

    @functools.partial(
        pl.kernel, mesh=mesh,
        out_type=jax.ShapeDtypeStruct((bn, c2), jnp.float32),
        scratch_types=[
            pltpu.VMEM((p,), jnp.int32),
            pltpu.VMEM((p,), jnp.int32),
            pltpu.VMEM((p,), jnp.int32),
            pltpu.VMEM((p, 48), jnp.float32),
            pltpu.VMEM((p, 128), jnp.float32),
            pltpu.VMEM((p, 128), jnp.float32),
            pltpu.VMEM((p, 128), jnp.float32),
            pltpu.VMEM((p, c2), jnp.float32),
            pltpu.SemaphoreType.DMA,
        ],
    )
    def sck(table_h, idx_h, w_h, out_h, i0, i1, i2, wv, r0, r1, r2, ot, sem):
        wid = lax.axis_index("s") * nc + lax.axis_index("c")

        def chunk_body(ci, carry):
            pbase = wid * ppw + ci * p
            b = pbase // n
            off = pbase % n
            pltpu.sync_copy(idx_h.at[b, 0, pl.ds(off, p)], i0)
            pltpu.sync_copy(idx_h.at[b, 1, pl.ds(off, p)], i1)
            pltpu.sync_copy(idx_h.at[b, 2, pl.ds(off, p)], i2)
            pltpu.sync_copy(w_h.at[pl.ds(pbase, p), :], wv)
            d0 = pltpu.async_copy(table_h.at[i0], r0, sem)
            d1 = pltpu.async_copy(table_h.at[i1], r1, sem)
            d2 = pltpu.async_copy(table_h.at[i2], r2, sem)
            d0.wait()
            d1.wait()
            d2.wait()

            def pt_body(i, carry2):
                wv0 = wv[i, pl.ds(0, 16)]
                wv1 = wv[i, pl.ds(16, 16)]
                wv2 = wv[i, pl.ds(32, 16)]
                for c in range(nch):
                    sl = pl.ds(c * 16, 16)
                    ot[i, sl] = (wv0 * r0[i, sl] + wv1 * r1[i, sl]
                                 + wv2 * r2[i, sl])
                return carry2

            lax.fori_loop(0, p, pt_body, 0)
            pltpu.sync_copy(ot, out_h.at[pl.ds(pbase, p), :])
            return carry

        lax.fori_loop(0, nchunks, chunk_body, 0)

    return sck(table, idxg, wexp)


def _pass_mlp1(it_ref, p1_ref, w1_ref, b1_ref, y1_ref, st_ref):
    @pl.when(pl.program_id(0) == 0)
    def _init():
        st_ref[...] = jnp.zeros_like(st_ref)

    xcat = jnp.concatenate([it_ref[...], p1_ref[...]], axis=1)
    y1 = jnp.dot(xcat, w1_ref[...],
                 preferred_element_type=jnp.float32) + b1_ref[...]
    y1_ref[...] = y1
    st_ref[0:1, :] += jnp.sum(y1, axis=0, keepdims=True)
    st_ref[1:2, :] += jnp.sum(y1 * y1, axis=0, keepdims=True)


def _pass_mlp2(y1_ref, st1_ref, g1_ref, be1_ref, w2_ref, b2_ref,
               y2_ref, st2_ref, *, cnt):
    @pl.when(pl.program_id(0) == 0)
    def _init():
        st2_ref[...] = jnp.zeros_like(st2_ref)

    mean = st1_ref[0:1, :] / cnt
    var = st1_ref[1:2, :] / cnt - mean * mean
    inv = g1_ref[...] * jax.lax.rsqrt(var + 1e-5)
    h = jnp.maximum((y1_ref[...] - mean) * inv + be1_ref[...], 0.0)
    y2 = jnp.dot(h, w2_ref[...],
                 preferred_element_type=jnp.float32) + b2_ref[...]
    y2_ref[...] = y2
    st2_ref[0:1, :] += jnp.sum(y2, axis=0, keepdims=True)
    st2_ref[1:2, :] += jnp.sum(y2 * y2, axis=0, keepdims=True)


def _pass_out(y2_ref, st2_ref, g2_ref, be2_ref, out_ref, *, cnt):
    mean = st2_ref[0:1, :] / cnt
    var = st2_ref[1:2, :] / cnt - mean * mean
    inv = g2_ref[...] * jax.lax.rsqrt(var + 1e-5)
    out_ref[...] = jnp.maximum((y2_ref[...] - mean) * inv + be2_ref[...], 0.0)


@jax.jit
def kernel(xyz1, xyz2, points1, points2, W1, b1, g1, be1, W2, b2, g2, be2):
    b_, n, _ = xyz1.shape
    m = xyz2.shape[1]
    c1 = points1.shape[1]
    c2 = points2.shape[1]
    h1 = W1.shape[0]
    h2 = W2.shape[0]
    tn = 256 if n % 256 == 0 else 128
    nt = n // tn
    bn = b_ * n
    ng = bn // tn
    cnt = float(bn)

    xyz1p = jnp.zeros((b_, 8, n), jnp.float32)
    xyz1p = xyz1p.at[:, 0:3, :].set(jnp.swapaxes(xyz1, 1, 2))

    idxg, wts = pl.pallas_call(
        functools.partial(_pass_nn, m=m, tn=tn),
        grid=(b_, nt),
        in_specs=[
            pl.BlockSpec((1, 8, tn), lambda b, i: (b, 0, i)),
            pl.BlockSpec((1, m, 3), lambda b, i: (b, 0, 0)),
        ],
        out_specs=[
            pl.BlockSpec((1, 8, tn), lambda b, i: (b, 0, i)),
            pl.BlockSpec((1, 8, tn), lambda b, i: (b, 0, i)),
        ],
        out_shape=[
            jax.ShapeDtypeStruct((b_, 8, n), jnp.int32),
            jax.ShapeDtypeStruct((b_, 8, n), jnp.float32),
        ],
    )(xyz1p, xyz2)

    table = jnp.zeros((b_ * m, 128), jnp.float32)
    table = table.at[:, 0:c2].set(jnp.swapaxes(points2, 1, 2).reshape(b_ * m, c2))
    w_pm = jnp.swapaxes(wts[:, 0:3, :], 1, 2).reshape(bn, 3)
    wexp = jnp.repeat(w_pm, 16, axis=1)

    interp = _sc_interp(table, idxg, wexp, bn, n, c2)

    p1t = jnp.swapaxes(points1, 1, 2).reshape(bn, c1)
    row = lambda v: v.reshape(1, -1)

    y1, st1 = pl.pallas_call(
        _pass_mlp1,
        grid=(ng,),
        in_specs=[
            pl.BlockSpec((tn, c2), lambda g: (g, 0)),
            pl.BlockSpec((tn, c1), lambda g: (g, 0)),
            pl.BlockSpec((c1 + c2, h1), lambda g: (0, 0)),
            pl.BlockSpec((1, h1), lambda g: (0, 0)),
        ],
        out_specs=[
            pl.BlockSpec((tn, h1), lambda g: (g, 0)),
            pl.BlockSpec((8, h1), lambda g: (0, 0)),
        ],
        out_shape=[
            jax.ShapeDtypeStruct((bn, h1), jnp.float32),
            jax.ShapeDtypeStruct((8, h1), jnp.float32),
        ],
    )(interp, p1t, W1.T, row(b1))

    y2, st2 = pl.pallas_call(
        functools.partial(_pass_mlp2, cnt=cnt),
        grid=(ng,),
        in_specs=[
            pl.BlockSpec((tn, h1), lambda g: (g, 0)),
            pl.BlockSpec((8, h1), lambda g: (0, 0)),
            pl.BlockSpec((1, h1), lambda g: (0, 0)),
            pl.BlockSpec((1, h1), lambda g: (0, 0)),
            pl.BlockSpec((h1, h2), lambda g: (0, 0)),
            pl.BlockSpec((1, h2), lambda g: (0, 0)),
        ],
        out_specs=[
            pl.BlockSpec((tn, h2), lambda g: (g, 0)),
            pl.BlockSpec((8, h2), lambda g: (0, 0)),
        ],
        out_shape=[
            jax.ShapeDtypeStruct((bn, h2), jnp.float32),
            jax.ShapeDtypeStruct((8, h2), jnp.float32),
        ],
    )(y1, st1, row(g1), row(be1), W2.T, row(b2))

    out_pm = pl.pallas_call(
        functools.partial(_pass_out, cnt=cnt),
        grid=(ng,),
        in_specs=[
            pl.BlockSpec((tn, h2), lambda g: (g, 0)),
            pl.BlockSpec((8, h2), lambda g: (0, 0)),
            pl.BlockSpec((1, h2), lambda g: (0, 0)),
            pl.BlockSpec((1, h2), lambda g: (0, 0)),
        ],
        out_specs=pl.BlockSpec((tn, h2), lambda g: (g, 0)),
        out_shape=jax.ShapeDtypeStruct((bn, h2), jnp.float32),
    )(y2, st2, row(g2), row(be2))

    return jnp.swapaxes(out_pm.reshape(b_, n, h2), 1, 2)


# speedup vs baseline: 1.4774x; 1.4774x over previous
import functools

import jax
import jax.numpy as jnp
from jax import lax
from jax.experimental import pallas as pl
from jax.experimental.pallas import tpu as pltpu
from jax.experimental.pallas import tpu_sc as plsc


def _pass_nn(xyz1_ref, xyz2_ref, idx_ref, w_ref, *, m, tn):
    x = xyz1_ref[0]
    y = xyz2_ref[0]
    xb = x[0:3, :].astype(jnp.bfloat16)
    yb = y.astype(jnp.bfloat16)
    cross = jnp.dot(yb, xb, preferred_element_type=jnp.float32)
    sq1 = ((x[0:1, :] * x[0:1, :] + x[1:2, :] * x[1:2, :])
           + x[2:3, :] * x[2:3, :])
    sq2 = ((y[:, 0:1] * y[:, 0:1] + y[:, 1:2] * y[:, 1:2])
           + y[:, 2:3] * y[:, 2:3])
    d2 = (sq1 + sq2) - 2.0 * cross

    iota = jax.lax.broadcasted_iota(jnp.int32, (m, tn), 0)
    dst = []
    idx = []
    for k in range(3):
        mn = jnp.min(d2, axis=0, keepdims=True)
        im = jnp.argmin(d2, axis=0).astype(jnp.int32).reshape(1, tn)
        dst.append(mn)
        idx.append(im)
        if k < 2:
            d2 = jnp.where(iota == im, jnp.float32(3.4e38), d2)

    rec = [1.0 / jnp.maximum(d, 1e-10) for d in dst]
    norm = rec[0] + rec[1] + rec[2]
    base = pl.program_id(0) * m
    for k in range(3):
        idx_ref[0, k:k + 1, :] = idx[k] + base
        w_ref[0, k:k + 1, :] = rec[k] / norm


def _sc_interp(table, idxg, wexp, bn, n, c2):
    info = plsc.get_sparse_core_info()
    nc = info.num_cores
    nw = nc * info.num_subcores
    ppw = bn // nw
    p = min(128, ppw)
    nchunks = ppw // p
    nch = c2 // 16
    mesh = plsc.VectorSubcoreMesh(core_axis_name="c", subcore_axis_name="s")

    @functools.partial(
        pl.kernel, mesh=mesh,
        out_type=jax.ShapeDtypeStruct((bn, c2), jnp.float32),
        scratch_types=[
            pltpu.VMEM((p,), jnp.int32),
            pltpu.VMEM((p,), jnp.int32),
            pltpu.VMEM((p,), jnp.int32),
            pltpu.VMEM((p, 48), jnp.float32),
            pltpu.VMEM((p, 128), jnp.float32),
            pltpu.VMEM((p, 128), jnp.float32),
            pltpu.VMEM((p, 128), jnp.float32),
            pltpu.VMEM((p, c2), jnp.float32),
            pltpu.SemaphoreType.DMA,
        ],
    )
    def sck(table_h, idx_h, w_h, out_h, i0, i1, i2, wv, r0, r1, r2, ot, sem):
        wid = lax.axis_index("s") * nc + lax.axis_index("c")

        def chunk_body(ci, carry):
            pbase = wid * ppw + ci * p
            b = pbase // n
            off = pbase % n
            pltpu.sync_copy(idx_h.at[b, 0, pl.ds(off, p)], i0)
            pltpu.sync_copy(idx_h.at[b, 1, pl.ds(off, p)], i1)
            pltpu.sync_copy(idx_h.at[b, 2, pl.ds(off, p)], i2)
            pltpu.sync_copy(w_h.at[pl.ds(pbase, p), :], wv)
            d0 = pltpu.async_copy(table_h.at[i0], r0, sem)
            d1 = pltpu.async_copy(table_h.at[i1], r1, sem)
            d2 = pltpu.async_copy(table_h.at[i2], r2, sem)
            d0.wait()
            d1.wait()
            d2.wait()

            def pt_body(i, carry2):
                wv0 = wv[i, pl.ds(0, 16)]
                wv1 = wv[i, pl.ds(16, 16)]
                wv2 = wv[i, pl.ds(32, 16)]
                for c in range(nch):
                    sl = pl.ds(c * 16, 16)
                    ot[i, sl] = (wv0 * r0[i, sl] + wv1 * r1[i, sl]
                                 + wv2 * r2[i, sl])
                return carry2

            lax.fori_loop(0, p, pt_body, 0)
            pltpu.sync_copy(ot, out_h.at[pl.ds(pbase, p), :])
            return carry

        lax.fori_loop(0, nchunks, chunk_body, 0)

    return sck(table, idxg, wexp)


def _pass_mlp1(it_ref, p1_ref, w1_ref, b1_ref, y1_ref, st_ref):
    @pl.when(pl.program_id(0) == 0)
    def _init():
        st_ref[...] = jnp.zeros_like(st_ref)

    xcat = jnp.concatenate([it_ref[...], p1_ref[...]], axis=1)
    y1 = jnp.dot(xcat, w1_ref[...],
                 preferred_element_type=jnp.float32) + b1_ref[...]
    y1_ref[...] = y1
    st_ref[0:1, :] += jnp.sum(y1, axis=0, keepdims=True)
    st_ref[1:2, :] += jnp.sum(y1 * y1, axis=0, keepdims=True)


def _pass_mlp2(y1_ref, st1_ref, g1_ref, be1_ref, w2_ref, b2_ref,
               y2_ref, st2_ref, *, cnt):
    @pl.when(pl.program_id(0) == 0)
    def _init():
        st2_ref[...] = jnp.zeros_like(st2_ref)

    mean = st1_ref[0:1, :] / cnt
    var = st1_ref[1:2, :] / cnt - mean * mean
    inv = g1_ref[...] * jax.lax.rsqrt(var + 1e-5)
    h = jnp.maximum((y1_ref[...] - mean) * inv + be1_ref[...], 0.0)
    y2 = jnp.dot(h, w2_ref[...],
                 preferred_element_type=jnp.float32) + b2_ref[...]
    y2_ref[...] = y2
    st2_ref[0:1, :] += jnp.sum(y2, axis=0, keepdims=True)
    st2_ref[1:2, :] += jnp.sum(y2 * y2, axis=0, keepdims=True)


def _pass_out(y2_ref, st2_ref, g2_ref, be2_ref, out_ref, *, cnt):
    mean = st2_ref[0:1, :] / cnt
    var = st2_ref[1:2, :] / cnt - mean * mean
    inv = g2_ref[...] * jax.lax.rsqrt(var + 1e-5)
    out_ref[...] = jnp.maximum((y2_ref[...] - mean) * inv + be2_ref[...], 0.0)


@jax.jit
def kernel(xyz1, xyz2, points1, points2, W1, b1, g1, be1, W2, b2, g2, be2):
    b_, n, _ = xyz1.shape
    m = xyz2.shape[1]
    c1 = points1.shape[1]
    c2 = points2.shape[1]
    h1 = W1.shape[0]
    h2 = W2.shape[0]
    tn = 512 if n % 512 == 0 else (256 if n % 256 == 0 else 128)
    nt = n // tn
    bn = b_ * n
    ng = bn // tn
    cnt = float(bn)

    xyz1p = jnp.zeros((b_, 8, n), jnp.float32)
    xyz1p = xyz1p.at[:, 0:3, :].set(jnp.swapaxes(xyz1, 1, 2))

    idxg, wts = pl.pallas_call(
        functools.partial(_pass_nn, m=m, tn=tn),
        grid=(b_, nt),
        in_specs=[
            pl.BlockSpec((1, 8, tn), lambda b, i: (b, 0, i)),
            pl.BlockSpec((1, m, 3), lambda b, i: (b, 0, 0)),
        ],
        out_specs=[
            pl.BlockSpec((1, 8, tn), lambda b, i: (b, 0, i)),
            pl.BlockSpec((1, 8, tn), lambda b, i: (b, 0, i)),
        ],
        out_shape=[
            jax.ShapeDtypeStruct((b_, 8, n), jnp.int32),
            jax.ShapeDtypeStruct((b_, 8, n), jnp.float32),
        ],
    )(xyz1p, xyz2)

    table = jnp.zeros((b_ * m, 128), jnp.float32)
    table = table.at[:, 0:c2].set(jnp.swapaxes(points2, 1, 2).reshape(b_ * m, c2))
    w_pm = jnp.swapaxes(wts[:, 0:3, :], 1, 2).reshape(bn, 3)
    wexp = jnp.repeat(w_pm, 16, axis=1)

    interp = _sc_interp(table, idxg, wexp, bn, n, c2)

    p1t = jnp.swapaxes(points1, 1, 2).reshape(bn, c1)
    row = lambda v: v.reshape(1, -1)

    y1, st1 = pl.pallas_call(
        _pass_mlp1,
        grid=(ng,),
        in_specs=[
            pl.BlockSpec((tn, c2), lambda g: (g, 0)),
            pl.BlockSpec((tn, c1), lambda g: (g, 0)),
            pl.BlockSpec((c1 + c2, h1), lambda g: (0, 0)),
            pl.BlockSpec((1, h1), lambda g: (0, 0)),
        ],
        out_specs=[
            pl.BlockSpec((tn, h1), lambda g: (g, 0)),
            pl.BlockSpec((8, h1), lambda g: (0, 0)),
        ],
        out_shape=[
            jax.ShapeDtypeStruct((bn, h1), jnp.float32),
            jax.ShapeDtypeStruct((8, h1), jnp.float32),
        ],
    )(interp, p1t, W1.T, row(b1))

    y2, st2 = pl.pallas_call(
        functools.partial(_pass_mlp2, cnt=cnt),
        grid=(ng,),
        in_specs=[
            pl.BlockSpec((tn, h1), lambda g: (g, 0)),
            pl.BlockSpec((8, h1), lambda g: (0, 0)),
            pl.BlockSpec((1, h1), lambda g: (0, 0)),
            pl.BlockSpec((1, h1), lambda g: (0, 0)),
            pl.BlockSpec((h1, h2), lambda g: (0, 0)),
            pl.BlockSpec((1, h2), lambda g: (0, 0)),
        ],
        out_specs=[
            pl.BlockSpec((tn, h2), lambda g: (g, 0)),
            pl.BlockSpec((8, h2), lambda g: (0, 0)),
        ],
        out_shape=[
            jax.ShapeDtypeStruct((bn, h2), jnp.float32),
            jax.ShapeDtypeStruct((8, h2), jnp.float32),
        ],
    )(y1, st1, row(g1), row(be1), W2.T, row(b2))

    out_pm = pl.pallas_call(
        functools.partial(_pass_out, cnt=cnt),
        grid=(ng,),
        in_specs=[
            pl.BlockSpec((tn, h2), lambda g: (g, 0)),
            pl.BlockSpec((8, h2), lambda g: (0, 0)),
            pl.BlockSpec((1, h2), lambda g: (0, 0)),
            pl.BlockSpec((1, h2), lambda g: (0, 0)),
        ],
        out_specs=pl.BlockSpec((tn, h2), lambda g: (g, 0)),
        out_shape=jax.ShapeDtypeStruct((bn, h2), jnp.float32),
    )(y2, st2, row(g2), row(be2))

    return jnp.swapaxes(out_pm.reshape(b_, n, h2), 1, 2)


# speedup vs baseline: 1.5345x; 1.0386x over previous
import functools

import jax
import jax.numpy as jnp
from jax import lax
from jax.experimental import pallas as pl
from jax.experimental.pallas import tpu as pltpu
from jax.experimental.pallas import tpu_sc as plsc


def _pass_nn(xyz1_ref, xyz2_ref, idx_ref, w_ref, *, m, tn, base):
    x = xyz1_ref[...]
    y = xyz2_ref[...]
    xb = x[0:3, :].astype(jnp.bfloat16)
    yb = y.astype(jnp.bfloat16)
    cross = jnp.dot(yb, xb, preferred_element_type=jnp.float32)
    sq1 = ((x[0:1, :] * x[0:1, :] + x[1:2, :] * x[1:2, :])
           + x[2:3, :] * x[2:3, :])
    sq2 = ((y[:, 0:1] * y[:, 0:1] + y[:, 1:2] * y[:, 1:2])
           + y[:, 2:3] * y[:, 2:3])
    d2 = (sq1 + sq2) - 2.0 * cross

    iota = jax.lax.broadcasted_iota(jnp.int32, (m, tn), 0)
    dst = []
    idx = []
    for k in range(3):
        mn = jnp.min(d2, axis=0, keepdims=True)
        im = jnp.argmin(d2, axis=0).astype(jnp.int32).reshape(1, tn)
        dst.append(mn)
        idx.append(im)
        if k < 2:
            d2 = jnp.where(iota == im, jnp.float32(3.4e38), d2)

    rec = [1.0 / jnp.maximum(d, 1e-10) for d in dst]
    norm = rec[0] + rec[1] + rec[2]
    for k in range(3):
        idx_ref[k:k + 1, :] = idx[k] + base
        w_ref[k:k + 1, :] = rec[k] / norm


def _sc_interp(table, idxg, wexp, npts, c2):
    info = plsc.get_sparse_core_info()
    nc = info.num_cores
    nw = nc * info.num_subcores
    ppw = npts // nw
    p = min(128, ppw)
    nchunks = ppw // p
    nch = c2 // 16
    mesh = plsc.VectorSubcoreMesh(core_axis_name="c", subcore_axis_name="s")

    @functools.partial(
        pl.kernel, mesh=mesh,
        out_type=jax.ShapeDtypeStruct((npts, c2), jnp.float32),
        scratch_types=[
            pltpu.VMEM((p,), jnp.int32),
            pltpu.VMEM((p,), jnp.int32),
            pltpu.VMEM((p,), jnp.int32),
            pltpu.VMEM((p, 48), jnp.float32),
            pltpu.VMEM((p, 128), jnp.float32),
            pltpu.VMEM((p, 128), jnp.float32),
            pltpu.VMEM((p, 128), jnp.float32),
            pltpu.VMEM((p, c2), jnp.float32),
            pltpu.SemaphoreType.DMA,
        ],
    )
    def sck(table_h, idx_h, w_h, out_h, i0, i1, i2, wv, r0, r1, r2, ot, sem):
        wid = lax.axis_index("s") * nc + lax.axis_index("c")

        def chunk_body(ci, carry):
            pbase = wid * ppw + ci * p
            pltpu.sync_copy(idx_h.at[0, pl.ds(pbase, p)], i0)
            pltpu.sync_copy(idx_h.at[1, pl.ds(pbase, p)], i1)
            pltpu.sync_copy(idx_h.at[2, pl.ds(pbase, p)], i2)
            pltpu.sync_copy(w_h.at[pl.ds(pbase, p), :], wv)
            d0 = pltpu.async_copy(table_h.at[i0], r0, sem)
            d1 = pltpu.async_copy(table_h.at[i1], r1, sem)
            d2 = pltpu.async_copy(table_h.at[i2], r2, sem)
            d0.wait()
            d1.wait()
            d2.wait()

            def pt_body(i, carry2):
                wv0 = wv[i, pl.ds(0, 16)]
                wv1 = wv[i, pl.ds(16, 16)]
                wv2 = wv[i, pl.ds(32, 16)]
                for c in range(nch):
                    sl = pl.ds(c * 16, 16)
                    ot[i, sl] = (wv0 * r0[i, sl] + wv1 * r1[i, sl]
                                 + wv2 * r2[i, sl])
                return carry2

            lax.fori_loop(0, p, pt_body, 0)
            pltpu.sync_copy(ot, out_h.at[pl.ds(pbase, p), :])
            return carry

        lax.fori_loop(0, nchunks, chunk_body, 0)

    return sck(table, idxg, wexp)


def _pass_mlp1(it_ref, p1_ref, w1_ref, b1_ref, y1_ref, st_ref):
    @pl.when(pl.program_id(0) == 0)
    def _init():
        st_ref[...] = jnp.zeros_like(st_ref)

    xcat = jnp.concatenate([it_ref[...], p1_ref[...]], axis=1)
    y1 = jnp.dot(xcat, w1_ref[...],
                 preferred_element_type=jnp.float32) + b1_ref[...]
    y1_ref[...] = y1
    st_ref[0:1, :] += jnp.sum(y1, axis=0, keepdims=True)
    st_ref[1:2, :] += jnp.sum(y1 * y1, axis=0, keepdims=True)


def _pass_mlp2(y1_ref, st1_ref, g1_ref, be1_ref, w2_ref, b2_ref,
               y2_ref, st2_ref, *, cnt):
    @pl.when(pl.program_id(0) == 0)
    def _init():
        st2_ref[...] = jnp.zeros_like(st2_ref)

    mean = st1_ref[0:1, :] / cnt
    var = st1_ref[1:2, :] / cnt - mean * mean
    inv = g1_ref[...] * jax.lax.rsqrt(var + 1e-5)
    h = jnp.maximum((y1_ref[...] - mean) * inv + be1_ref[...], 0.0)
    y2 = jnp.dot(h, w2_ref[...],
                 preferred_element_type=jnp.float32) + b2_ref[...]
    y2_ref[...] = y2
    st2_ref[0:1, :] += jnp.sum(y2, axis=0, keepdims=True)
    st2_ref[1:2, :] += jnp.sum(y2 * y2, axis=0, keepdims=True)


def _pass_out(y2_ref, st2_ref, g2_ref, be2_ref, out_ref, *, cnt):
    mean = st2_ref[0:1, :] / cnt
    var = st2_ref[1:2, :] / cnt - mean * mean
    inv = g2_ref[...] * jax.lax.rsqrt(var + 1e-5)
    out_ref[...] = jnp.maximum((y2_ref[...] - mean) * inv + be2_ref[...], 0.0)


@jax.jit
def kernel(xyz1, xyz2, points1, points2, W1, b1, g1, be1, W2, b2, g2, be2):
    b_, n, _ = xyz1.shape
    m = xyz2.shape[1]
    c1 = points1.shape[1]
    c2 = points2.shape[1]
    h1 = W1.shape[0]
    h2 = W2.shape[0]
    tn = 512 if n % 512 == 0 else (256 if n % 256 == 0 else 128)
    nt = n // tn
    bn = b_ * n
    ng = bn // tn
    cnt = float(bn)

    xyz1p = jnp.zeros((b_, 8, n), jnp.float32)
    xyz1p = xyz1p.at[:, 0:3, :].set(jnp.swapaxes(xyz1, 1, 2))

    table = jnp.zeros((b_ * m, 128), jnp.float32)
    table = table.at[:, 0:c2].set(jnp.swapaxes(points2, 1, 2).reshape(b_ * m, c2))

    parts = []
    for b in range(b_):
        idxg, wts = pl.pallas_call(
            functools.partial(_pass_nn, m=m, tn=tn, base=b * m),
            grid=(nt,),
            in_specs=[
                pl.BlockSpec((8, tn), lambda i: (0, i)),
                pl.BlockSpec((m, 3), lambda i: (0, 0)),
            ],
            out_specs=[
                pl.BlockSpec((8, tn), lambda i: (0, i)),
                pl.BlockSpec((8, tn), lambda i: (0, i)),
            ],
            out_shape=[
                jax.ShapeDtypeStruct((8, n), jnp.int32),
                jax.ShapeDtypeStruct((8, n), jnp.float32),
            ],
        )(xyz1p[b], xyz2[b])
        wexp = jnp.repeat(jnp.swapaxes(wts[0:3, :], 0, 1), 16, axis=1)
        parts.append(_sc_interp(table, idxg, wexp, n, c2))

    interp = jnp.concatenate(parts, axis=0)

    p1t = jnp.swapaxes(points1, 1, 2).reshape(bn, c1)
    row = lambda v: v.reshape(1, -1)

    y1, st1 = pl.pallas_call(
        _pass_mlp1,
        grid=(ng,),
        in_specs=[
            pl.BlockSpec((tn, c2), lambda g: (g, 0)),
            pl.BlockSpec((tn, c1), lambda g: (g, 0)),
            pl.BlockSpec((c1 + c2, h1), lambda g: (0, 0)),
            pl.BlockSpec((1, h1), lambda g: (0, 0)),
        ],
        out_specs=[
            pl.BlockSpec((tn, h1), lambda g: (g, 0)),
            pl.BlockSpec((8, h1), lambda g: (0, 0)),
        ],
        out_shape=[
            jax.ShapeDtypeStruct((bn, h1), jnp.float32),
            jax.ShapeDtypeStruct((8, h1), jnp.float32),
        ],
    )(interp, p1t, W1.T, row(b1))

    y2, st2 = pl.pallas_call(
        functools.partial(_pass_mlp2, cnt=cnt),
        grid=(ng,),
        in_specs=[
            pl.BlockSpec((tn, h1), lambda g: (g, 0)),
            pl.BlockSpec((8, h1), lambda g: (0, 0)),
            pl.BlockSpec((1, h1), lambda g: (0, 0)),
            pl.BlockSpec((1, h1), lambda g: (0, 0)),
            pl.BlockSpec((h1, h2), lambda g: (0, 0)),
            pl.BlockSpec((1, h2), lambda g: (0, 0)),
        ],
        out_specs=[
            pl.BlockSpec((tn, h2), lambda g: (g, 0)),
            pl.BlockSpec((8, h2), lambda g: (0, 0)),
        ],
        out_shape=[
            jax.ShapeDtypeStruct((bn, h2), jnp.float32),
            jax.ShapeDtypeStruct((8, h2), jnp.float32),
        ],
    )(y1, st1, row(g1), row(be1), W2.T, row(b2))

    out_pm = pl.pallas_call(
        functools.partial(_pass_out, cnt=cnt),
        grid=(ng,),
        in_specs=[
            pl.BlockSpec((tn, h2), lambda g: (g, 0)),
            pl.BlockSpec((8, h2), lambda g: (0, 0)),
            pl.BlockSpec((1, h2), lambda g: (0, 0)),
            pl.BlockSpec((1, h2), lambda g: (0, 0)),
        ],
        out_specs=pl.BlockSpec((tn, h2), lambda g: (g, 0)),
        out_shape=jax.ShapeDtypeStruct((bn, h2), jnp.float32),
    )(y2, st2, row(g2), row(be2))

    return jnp.swapaxes(out_pm.reshape(b_, n, h2), 1, 2)


# speedup vs baseline: 1.5557x; 1.0138x over previous
import functools

import jax
import jax.numpy as jnp
from jax import lax
from jax.experimental import pallas as pl
from jax.experimental.pallas import tpu as pltpu
from jax.experimental.pallas import tpu_sc as plsc


def _pass_nn(xyz1_ref, xyz2_ref, idx_ref, w_ref, *, m, tn, base):
    x = xyz1_ref[...]
    y = xyz2_ref[...]
    xb = x[0:3, :].astype(jnp.bfloat16)
    yb2 = (y + y).astype(jnp.bfloat16)
    cross2 = jnp.dot(yb2, xb, preferred_element_type=jnp.float32)
    sq1 = ((x[0:1, :] * x[0:1, :] + x[1:2, :] * x[1:2, :])
           + x[2:3, :] * x[2:3, :])
    sq2 = ((y[:, 0:1] * y[:, 0:1] + y[:, 1:2] * y[:, 1:2])
           + y[:, 2:3] * y[:, 2:3])
    d2 = (sq1 + sq2) - cross2

    iota = jax.lax.broadcasted_iota(jnp.int32, (m, tn), 0)
    dst = []
    idx = []
    for k in range(3):
        mn = jnp.min(d2, axis=0, keepdims=True)
        im = jnp.argmin(d2, axis=0).astype(jnp.int32).reshape(1, tn)
        dst.append(mn)
        idx.append(im)
        if k < 2:
            d2 = jnp.where(iota == im, jnp.float32(3.4e38), d2)

    rec = [1.0 / jnp.maximum(d, 1e-10) for d in dst]
    norm = rec[0] + rec[1] + rec[2]
    for k in range(3):
        idx_ref[k:k + 1, :] = idx[k] + base
        w_ref[k:k + 1, :] = rec[k] / norm


def _sc_interp(table, idxg, wexp, npts, c2):
    info = plsc.get_sparse_core_info()
    nc = info.num_cores
    nw = nc * info.num_subcores
    ppw = npts // nw
    p = min(64, ppw)
    nchunks = ppw // p
    nch = c2 // 16
    mesh = plsc.VectorSubcoreMesh(core_axis_name="c", subcore_axis_name="s")

    buf = lambda: [
        pltpu.VMEM((p,), jnp.int32),
        pltpu.VMEM((p,), jnp.int32),
        pltpu.VMEM((p,), jnp.int32),
        pltpu.VMEM((p, 48), jnp.float32),
        pltpu.VMEM((p, 128), jnp.float32),
        pltpu.VMEM((p, 128), jnp.float32),
        pltpu.VMEM((p, 128), jnp.float32),
        pltpu.VMEM((p, c2), jnp.float32),
        pltpu.SemaphoreType.DMA,
        pltpu.SemaphoreType.DMA,
    ]

    @functools.partial(
        pl.kernel, mesh=mesh,
        out_type=jax.ShapeDtypeStruct((npts, c2), jnp.float32),
        scratch_types=buf() + buf(),
    )
    def sck(table_h, idx_h, w_h, out_h, *scr):
        wid = lax.axis_index("s") * nc + lax.axis_index("c")
        bufs = [scr[0:10], scr[10:20]]

        def stage_in(ci, bb):
            i0, i1, i2, wv = bb[0], bb[1], bb[2], bb[3]
            pbase = wid * ppw + ci * p
            pltpu.sync_copy(idx_h.at[0, pl.ds(pbase, p)], i0)
            pltpu.sync_copy(idx_h.at[1, pl.ds(pbase, p)], i1)
            pltpu.sync_copy(idx_h.at[2, pl.ds(pbase, p)], i2)
            pltpu.sync_copy(w_h.at[pl.ds(pbase, p), :], wv)
            return [pltpu.async_copy(table_h.at[bb[k]], bb[4 + k], bb[8])
                    for k in range(3)]

        def compute(ci, bb, handles):
            wv, r0, r1, r2, ot = bb[3], bb[4], bb[5], bb[6], bb[7]
            for h in handles:
                h.wait()

            def pt_body(i, carry2):
                wv0 = wv[i, pl.ds(0, 16)]
                wv1 = wv[i, pl.ds(16, 16)]
                wv2 = wv[i, pl.ds(32, 16)]
                for c in range(nch):
                    sl = pl.ds(c * 16, 16)
                    ot[i, sl] = (wv0 * r0[i, sl] + wv1 * r1[i, sl]
                                 + wv2 * r2[i, sl])
                return carry2

            lax.fori_loop(0, p, pt_body, 0)
            pbase = wid * ppw + ci * p
            pltpu.sync_copy(ot, out_h.at[pl.ds(pbase, p), :])

        handles = stage_in(0, bufs[0])
        for ci in range(nchunks):
            nxt = None
            if ci + 1 < nchunks:
                nxt = stage_in(ci + 1, bufs[(ci + 1) % 2])
            compute(ci, bufs[ci % 2], handles)
            handles = nxt

    return sck(table, idxg, wexp)


def _pass_mlp1(it_ref, p1_ref, w1_ref, b1_ref, y1_ref, st_ref):
    @pl.when(pl.program_id(0) == 0)
    def _init():
        st_ref[...] = jnp.zeros_like(st_ref)

    xcat = jnp.concatenate([it_ref[...], p1_ref[...]], axis=1)
    y1 = jnp.dot(xcat, w1_ref[...],
                 preferred_element_type=jnp.float32) + b1_ref[...]
    y1_ref[...] = y1
    st_ref[0:1, :] += jnp.sum(y1, axis=0, keepdims=True)
    st_ref[1:2, :] += jnp.sum(y1 * y1, axis=0, keepdims=True)


def _pass_mlp2(y1_ref, st1_ref, g1_ref, be1_ref, w2_ref, b2_ref,
               y2_ref, st2_ref, *, cnt):
    @pl.when(pl.program_id(0) == 0)
    def _init():
        st2_ref[...] = jnp.zeros_like(st2_ref)

    mean = st1_ref[0:1, :] / cnt
    var = st1_ref[1:2, :] / cnt - mean * mean
    inv = g1_ref[...] * jax.lax.rsqrt(var + 1e-5)
    h = jnp.maximum((y1_ref[...] - mean) * inv + be1_ref[...], 0.0)
    y2 = jnp.dot(h, w2_ref[...],
                 preferred_element_type=jnp.float32) + b2_ref[...]
    y2_ref[...] = y2
    st2_ref[0:1, :] += jnp.sum(y2, axis=0, keepdims=True)
    st2_ref[1:2, :] += jnp.sum(y2 * y2, axis=0, keepdims=True)


def _pass_out(y2_ref, st2_ref, g2_ref, be2_ref, out_ref, *, cnt):
    mean = st2_ref[0:1, :] / cnt
    var = st2_ref[1:2, :] / cnt - mean * mean
    inv = g2_ref[...] * jax.lax.rsqrt(var + 1e-5)
    out_ref[...] = jnp.maximum((y2_ref[...] - mean) * inv + be2_ref[...], 0.0)


@jax.jit
def kernel(xyz1, xyz2, points1, points2, W1, b1, g1, be1, W2, b2, g2, be2):
    b_, n, _ = xyz1.shape
    m = xyz2.shape[1]
    c1 = points1.shape[1]
    c2 = points2.shape[1]
    h1 = W1.shape[0]
    h2 = W2.shape[0]
    tn = 512 if n % 512 == 0 else (256 if n % 256 == 0 else 128)
    nt = n // tn
    bn = b_ * n
    ng = bn // tn
    cnt = float(bn)

    xyz1p = jnp.zeros((b_, 8, n), jnp.float32)
    xyz1p = xyz1p.at[:, 0:3, :].set(jnp.swapaxes(xyz1, 1, 2))

    table = jnp.zeros((b_ * m, 128), jnp.float32)
    table = table.at[:, 0:c2].set(jnp.swapaxes(points2, 1, 2).reshape(b_ * m, c2))

    parts = []
    for b in range(b_):
        idxg, wts = pl.pallas_call(
            functools.partial(_pass_nn, m=m, tn=tn, base=b * m),
            grid=(nt,),
            in_specs=[
                pl.BlockSpec((8, tn), lambda i: (0, i)),
                pl.BlockSpec((m, 3), lambda i: (0, 0)),
            ],
            out_specs=[
                pl.BlockSpec((8, tn), lambda i: (0, i)),
                pl.BlockSpec((8, tn), lambda i: (0, i)),
            ],
            out_shape=[
                jax.ShapeDtypeStruct((8, n), jnp.int32),
                jax.ShapeDtypeStruct((8, n), jnp.float32),
            ],
        )(xyz1p[b], xyz2[b])
        wexp = jnp.repeat(jnp.swapaxes(wts[0:3, :], 0, 1), 16, axis=1)
        parts.append(_sc_interp(table, idxg, wexp, n, c2))

    interp = jnp.concatenate(parts, axis=0)

    p1t = jnp.swapaxes(points1, 1, 2).reshape(bn, c1)
    row = lambda v: v.reshape(1, -1)

    y1, st1 = pl.pallas_call(
        _pass_mlp1,
        grid=(ng,),
        in_specs=[
            pl.BlockSpec((tn, c2), lambda g: (g, 0)),
            pl.BlockSpec((tn, c1), lambda g: (g, 0)),
            pl.BlockSpec((c1 + c2, h1), lambda g: (0, 0)),
            pl.BlockSpec((1, h1), lambda g: (0, 0)),
        ],
        out_specs=[
            pl.BlockSpec((tn, h1), lambda g: (g, 0)),
            pl.BlockSpec((8, h1), lambda g: (0, 0)),
        ],
        out_shape=[
            jax.ShapeDtypeStruct((bn, h1), jnp.float32),
            jax.ShapeDtypeStruct((8, h1), jnp.float32),
        ],
    )(interp, p1t, W1.T, row(b1))

    y2, st2 = pl.pallas_call(
        functools.partial(_pass_mlp2, cnt=cnt),
        grid=(ng,),
        in_specs=[
            pl.BlockSpec((tn, h1), lambda g: (g, 0)),
            pl.BlockSpec((8, h1), lambda g: (0, 0)),
            pl.BlockSpec((1, h1), lambda g: (0, 0)),
            pl.BlockSpec((1, h1), lambda g: (0, 0)),
            pl.BlockSpec((h1, h2), lambda g: (0, 0)),
            pl.BlockSpec((1, h2), lambda g: (0, 0)),
        ],
        out_specs=[
            pl.BlockSpec((tn, h2), lambda g: (g, 0)),
            pl.BlockSpec((8, h2), lambda g: (0, 0)),
        ],
        out_shape=[
            jax.ShapeDtypeStruct((bn, h2), jnp.float32),
            jax.ShapeDtypeStruct((8, h2), jnp.float32),
        ],
    )(y1, st1, row(g1), row(be1), W2.T, row(b2))

    out_pm = pl.pallas_call(
        functools.partial(_pass_out, cnt=cnt),
        grid=(ng,),
        in_specs=[
            pl.BlockSpec((tn, h2), lambda g: (g, 0)),
            pl.BlockSpec((8, h2), lambda g: (0, 0)),
            pl.BlockSpec((1, h2), lambda g: (0, 0)),
            pl.BlockSpec((1, h2), lambda g: (0, 0)),
        ],
        out_specs=pl.BlockSpec((tn, h2), lambda g: (g, 0)),
        out_shape=jax.ShapeDtypeStruct((bn, h2), jnp.float32),
    )(y2, st2, row(g2), row(be2))

    return jnp.swapaxes(out_pm.reshape(b_, n, h2), 1, 2)


# speedup vs baseline: 1.5572x; 1.0010x over previous
import functools

import jax
import jax.numpy as jnp
from jax import lax
from jax.experimental import pallas as pl
from jax.experimental.pallas import tpu as pltpu
from jax.experimental.pallas import tpu_sc as plsc


def _pass_nn(xyz1_ref, xyz2_ref, idx_ref, w_ref, *, m, tn, base):
    x = xyz1_ref[...]
    y = xyz2_ref[...]
    xb = x[0:3, :].astype(jnp.bfloat16)
    yb2 = (y + y).astype(jnp.bfloat16)
    cross2 = jnp.dot(yb2, xb, preferred_element_type=jnp.float32)
    sq1 = ((x[0:1, :] * x[0:1, :] + x[1:2, :] * x[1:2, :])
           + x[2:3, :] * x[2:3, :])
    sq2 = ((y[:, 0:1] * y[:, 0:1] + y[:, 1:2] * y[:, 1:2])
           + y[:, 2:3] * y[:, 2:3])
    d2 = (sq1 + sq2) - cross2

    iota = jax.lax.broadcasted_iota(jnp.int32, (m, tn), 0)
    dst = []
    idx = []
    for k in range(3):
        mn = jnp.min(d2, axis=0, keepdims=True)
        im = jnp.argmin(d2, axis=0).astype(jnp.int32).reshape(1, tn)
        dst.append(mn)
        idx.append(im)
        if k < 2:
            d2 = jnp.where(iota == im, jnp.float32(3.4e38), d2)

    rec = [1.0 / jnp.maximum(d, 1e-10) for d in dst]
    norm = rec[0] + rec[1] + rec[2]
    for k in range(3):
        idx_ref[k:k + 1, :] = idx[k] + base
        w_ref[k:k + 1, :] = rec[k] / norm


def _sc_interp(table, idxg, wexp, npts, c2):
    info = plsc.get_sparse_core_info()
    nc = info.num_cores
    nw = nc * info.num_subcores
    ppw = npts // nw
    p = min(64, ppw)
    nchunks = ppw // p
    nch = c2 // 16
    mesh = plsc.VectorSubcoreMesh(core_axis_name="c", subcore_axis_name="s")

    buf = lambda: [
        pltpu.VMEM((p,), jnp.int32),
        pltpu.VMEM((p,), jnp.int32),
        pltpu.VMEM((p,), jnp.int32),
        pltpu.VMEM((p, 48), jnp.float32),
        pltpu.VMEM((p, 128), jnp.float32),
        pltpu.VMEM((p, 128), jnp.float32),
        pltpu.VMEM((p, 128), jnp.float32),
        pltpu.VMEM((p, c2), jnp.float32),
        pltpu.SemaphoreType.DMA,
        pltpu.SemaphoreType.DMA,
    ]

    @functools.partial(
        pl.kernel, mesh=mesh,
        compiler_params=pltpu.CompilerParams(use_tc_tiling_on_sc=True),
        out_type=jax.ShapeDtypeStruct((npts, c2), jnp.float32),
        scratch_types=buf() + buf(),
    )
    def sck(table_h, idx_h, w_h, out_h, *scr):
        wid = lax.axis_index("s") * nc + lax.axis_index("c")
        bufs = [scr[0:10], scr[10:20]]

        def stage_in(ci, bb):
            i0, i1, i2, wv = bb[0], bb[1], bb[2], bb[3]
            pbase = wid * ppw + ci * p
            pltpu.sync_copy(idx_h.at[0, pl.ds(pbase, p)], i0)
            pltpu.sync_copy(idx_h.at[1, pl.ds(pbase, p)], i1)
            pltpu.sync_copy(idx_h.at[2, pl.ds(pbase, p)], i2)
            pltpu.sync_copy(w_h.at[pl.ds(pbase, p), :], wv)
            return [pltpu.async_copy(table_h.at[bb[k]], bb[4 + k], bb[8])
                    for k in range(3)]

        def compute(ci, bb, handles):
            wv, r0, r1, r2, ot = bb[3], bb[4], bb[5], bb[6], bb[7]
            for h in handles:
                h.wait()

            def pt_body(i, carry2):
                wv0 = wv[i, pl.ds(0, 16)]
                wv1 = wv[i, pl.ds(16, 16)]
                wv2 = wv[i, pl.ds(32, 16)]
                for c in range(nch):
                    sl = pl.ds(c * 16, 16)
                    ot[i, sl] = (wv0 * r0[i, sl] + wv1 * r1[i, sl]
                                 + wv2 * r2[i, sl])
                return carry2

            lax.fori_loop(0, p, pt_body, 0)
            pbase = wid * ppw + ci * p
            pltpu.sync_copy(ot, out_h.at[pl.ds(pbase, p), :])

        handles = stage_in(0, bufs[0])
        for ci in range(nchunks):
            nxt = None
            if ci + 1 < nchunks:
                nxt = stage_in(ci + 1, bufs[(ci + 1) % 2])
            compute(ci, bufs[ci % 2], handles)
            handles = nxt

    return sck(table, idxg, wexp)


def _pass_mlp1(it_ref, p1_ref, w1_ref, b1_ref, y1_ref, st_ref):
    @pl.when(pl.program_id(0) == 0)
    def _init():
        st_ref[...] = jnp.zeros_like(st_ref)

    xcat = jnp.concatenate([it_ref[...], p1_ref[...]], axis=1)
    y1 = jnp.dot(xcat, w1_ref[...],
                 preferred_element_type=jnp.float32) + b1_ref[...]
    y1_ref[...] = y1
    st_ref[0:1, :] += jnp.sum(y1, axis=0, keepdims=True)
    st_ref[1:2, :] += jnp.sum(y1 * y1, axis=0, keepdims=True)


def _pass_mlp2(y1_ref, st1_ref, g1_ref, be1_ref, w2_ref, b2_ref,
               y2_ref, st2_ref, *, cnt):
    @pl.when(pl.program_id(0) == 0)
    def _init():
        st2_ref[...] = jnp.zeros_like(st2_ref)

    mean = st1_ref[0:1, :] / cnt
    var = st1_ref[1:2, :] / cnt - mean * mean
    inv = g1_ref[...] * jax.lax.rsqrt(var + 1e-5)
    h = jnp.maximum((y1_ref[...] - mean) * inv + be1_ref[...], 0.0)
    y2 = jnp.dot(h, w2_ref[...],
                 preferred_element_type=jnp.float32) + b2_ref[...]
    y2_ref[...] = y2
    st2_ref[0:1, :] += jnp.sum(y2, axis=0, keepdims=True)
    st2_ref[1:2, :] += jnp.sum(y2 * y2, axis=0, keepdims=True)


def _pass_out(y2_ref, st2_ref, g2_ref, be2_ref, out_ref, *, cnt):
    mean = st2_ref[0:1, :] / cnt
    var = st2_ref[1:2, :] / cnt - mean * mean
    inv = g2_ref[...] * jax.lax.rsqrt(var + 1e-5)
    out_ref[...] = jnp.maximum((y2_ref[...] - mean) * inv + be2_ref[...], 0.0)


@jax.jit
def kernel(xyz1, xyz2, points1, points2, W1, b1, g1, be1, W2, b2, g2, be2):
    b_, n, _ = xyz1.shape
    m = xyz2.shape[1]
    c1 = points1.shape[1]
    c2 = points2.shape[1]
    h1 = W1.shape[0]
    h2 = W2.shape[0]
    tn = 512 if n % 512 == 0 else (256 if n % 256 == 0 else 128)
    nt = n // tn
    bn = b_ * n
    ng = bn // tn
    cnt = float(bn)

    xyz1p = jnp.zeros((b_, 8, n), jnp.float32)
    xyz1p = xyz1p.at[:, 0:3, :].set(jnp.swapaxes(xyz1, 1, 2))

    table = jnp.zeros((b_ * m, 128), jnp.float32)
    table = table.at[:, 0:c2].set(jnp.swapaxes(points2, 1, 2).reshape(b_ * m, c2))

    parts = []
    for b in range(b_):
        idxg, wts = pl.pallas_call(
            functools.partial(_pass_nn, m=m, tn=tn, base=b * m),
            grid=(nt,),
            in_specs=[
                pl.BlockSpec((8, tn), lambda i: (0, i)),
                pl.BlockSpec((m, 3), lambda i: (0, 0)),
            ],
            out_specs=[
                pl.BlockSpec((8, tn), lambda i: (0, i)),
                pl.BlockSpec((8, tn), lambda i: (0, i)),
            ],
            out_shape=[
                jax.ShapeDtypeStruct((8, n), jnp.int32),
                jax.ShapeDtypeStruct((8, n), jnp.float32),
            ],
        )(xyz1p[b], xyz2[b])
        wexp = jnp.repeat(jnp.swapaxes(wts[0:3, :], 0, 1), 16, axis=1)
        parts.append(_sc_interp(table, idxg, wexp, n, c2))

    interp = jnp.concatenate(parts, axis=0)

    p1t = jnp.swapaxes(points1, 1, 2).reshape(bn, c1)
    row = lambda v: v.reshape(1, -1)

    y1, st1 = pl.pallas_call(
        _pass_mlp1,
        grid=(ng,),
        in_specs=[
            pl.BlockSpec((tn, c2), lambda g: (g, 0)),
            pl.BlockSpec((tn, c1), lambda g: (g, 0)),
            pl.BlockSpec((c1 + c2, h1), lambda g: (0, 0)),
            pl.BlockSpec((1, h1), lambda g: (0, 0)),
        ],
        out_specs=[
            pl.BlockSpec((tn, h1), lambda g: (g, 0)),
            pl.BlockSpec((8, h1), lambda g: (0, 0)),
        ],
        out_shape=[
            jax.ShapeDtypeStruct((bn, h1), jnp.float32),
            jax.ShapeDtypeStruct((8, h1), jnp.float32),
        ],
    )(interp, p1t, W1.T, row(b1))

    y2, st2 = pl.pallas_call(
        functools.partial(_pass_mlp2, cnt=cnt),
        grid=(ng,),
        in_specs=[
            pl.BlockSpec((tn, h1), lambda g: (g, 0)),
            pl.BlockSpec((8, h1), lambda g: (0, 0)),
            pl.BlockSpec((1, h1), lambda g: (0, 0)),
            pl.BlockSpec((1, h1), lambda g: (0, 0)),
            pl.BlockSpec((h1, h2), lambda g: (0, 0)),
            pl.BlockSpec((1, h2), lambda g: (0, 0)),
        ],
        out_specs=[
            pl.BlockSpec((tn, h2), lambda g: (g, 0)),
            pl.BlockSpec((8, h2), lambda g: (0, 0)),
        ],
        out_shape=[
            jax.ShapeDtypeStruct((bn, h2), jnp.float32),
            jax.ShapeDtypeStruct((8, h2), jnp.float32),
        ],
    )(y1, st1, row(g1), row(be1), W2.T, row(b2))

    out_pm = pl.pallas_call(
        functools.partial(_pass_out, cnt=cnt),
        grid=(ng,),
        in_specs=[
            pl.BlockSpec((tn, h2), lambda g: (g, 0)),
            pl.BlockSpec((8, h2), lambda g: (0, 0)),
            pl.BlockSpec((1, h2), lambda g: (0, 0)),
            pl.BlockSpec((1, h2), lambda g: (0, 0)),
        ],
        out_specs=pl.BlockSpec((tn, h2), lambda g: (g, 0)),
        out_shape=jax.ShapeDtypeStruct((bn, h2), jnp.float32),
    )(y2, st2, row(g2), row(be2))

    return jnp.swapaxes(out_pm.reshape(b_, n, h2), 1, 2)
